# baseline (speedup 1.0000x reference)
import functools

import jax
import jax.numpy as jnp
import numpy as np
from jax.experimental import pallas as pl
from jax.experimental.pallas import tpu as pltpu

_T = 512
_VOCAB = 100000

_EXP_GUMBEL = np.exp(
    np.asarray(jax.random.gumbel(jax.random.key(42), (_VOCAB,), jnp.float32),
               dtype=np.float64)
).astype(np.float32)


def _tc_body(t_ref, row_ref, eg_ref, out_ref):
    v = row_ref[0] * eg_ref[...]
    m = jnp.max(v)
    idx2 = jax.lax.broadcasted_iota(jnp.int32, v.shape, 1)
    amax = jnp.min(jnp.where(v == m, idx2, jnp.int32(2**31 - 1)))
    out_ref[...] = jnp.where(idx2 == amax, 0.0, 1000.0)


def kernel(weights, time):
    eg = jnp.asarray(_EXP_GUMBEL).reshape(1, _VOCAB)
    t_arr = jnp.asarray(time, jnp.int32).reshape(1)
    w3 = weights.reshape(_T, 1, _VOCAB)
    out = pl.pallas_call(
        _tc_body,
        grid_spec=pltpu.PrefetchScalarGridSpec(
            num_scalar_prefetch=1,
            grid=(1,),
            in_specs=[
                pl.BlockSpec((1, 1, _VOCAB), lambda i, t: (t[0], 0, 0)),
                pl.BlockSpec((1, _VOCAB), lambda i, t: (0, 0)),
            ],
            out_specs=pl.BlockSpec((1, _VOCAB), lambda i, t: (0, 0)),
        ),
        out_shape=jax.ShapeDtypeStruct((1, _VOCAB), jnp.float32),
    )(t_arr, w3, eg)
    return out.reshape(_VOCAB)


# speedup vs baseline: 2.4495x; 2.4495x over previous
import jax
import jax.numpy as jnp
import numpy as np
from jax.experimental import pallas as pl
from jax.experimental.pallas import tpu as pltpu

_T = 512
_VOCAB = 100000

_EXP_GUMBEL = np.exp(
    np.asarray(jax.random.gumbel(jax.random.key(42), (_VOCAB,), jnp.float32),
               dtype=np.float64)
).astype(np.float32)


def _tc_body(t_ref, rows_ref, eg_ref, out_ref):
    sub = t_ref[0] % 8
    v = rows_ref[...] * eg_ref[...]
    i0 = jax.lax.broadcasted_iota(jnp.int32, v.shape, 0)
    v = jnp.where(i0 == sub, v, -jnp.inf)
    m = jnp.max(v)
    idx = jax.lax.broadcasted_iota(jnp.int32, v.shape, 1)
    amax = jnp.min(jnp.where(v == m, idx, jnp.int32(2**31 - 1)))
    oidx = jax.lax.broadcasted_iota(jnp.int32, out_ref.shape, 1)
    out_ref[...] = jnp.where(oidx == amax, 0.0, 1000.0)


def kernel(weights, time):
    eg = jnp.asarray(_EXP_GUMBEL).reshape(1, _VOCAB)
    t_arr = jnp.asarray(time, jnp.int32).reshape(1)
    out = pl.pallas_call(
        _tc_body,
        grid_spec=pltpu.PrefetchScalarGridSpec(
            num_scalar_prefetch=1,
            grid=(1,),
            in_specs=[
                pl.BlockSpec((8, _VOCAB), lambda i, t: (t[0] // 8, 0)),
                pl.BlockSpec((1, _VOCAB), lambda i, t: (0, 0)),
            ],
            out_specs=pl.BlockSpec((1, _VOCAB), lambda i, t: (0, 0)),
        ),
        out_shape=jax.ShapeDtypeStruct((1, _VOCAB), jnp.float32),
    )(t_arr, weights, eg)
    return out.reshape(_VOCAB)
